# baseline (speedup 1.0000x reference)
import functools

import jax
import jax.numpy as jnp
import numpy as np
from jax import lax
from jax.experimental import pallas as pl
from jax.experimental.pallas import tpu as pltpu
from jax.experimental.pallas import tpu_sc as plsc

N = 50000
E = 800000
F = 64
NTYPES = 32
NBASIS = 8
NLAYERS = 4
RMAX = 5.0
AVG_NEIGH = 15.0
HID = 64

NC = 2
NS = 16
NW = NC * NS
CH = 128
GRP = 4
TPW = 200
NGRP = TPW // GRP
GROWS = GRP * CH
EPW = TPW * CH
EPAD = NW * EPW
NROWS_IDX = EPAD // CH

NPAD = 51200
FH = F // 2
STRIPE = NPAD // NS
DUMP = N + 1000
BN = 2048
NBLK_N = NPAD // BN
BEF = 4096
BE = 2048


def _silu(x):
    return x * jax.nn.sigmoid(x)


def _make_sc_gather(d, interpret=False):
    mesh = plsc.VectorSubcoreMesh(core_axis_name="c", subcore_axis_name="s",
                                  num_cores=NC, num_subcores=NS)

    def body(table_hbm, idx_hbm, out_hbm, idx_v, rows_v, gsem, ssem):
        c = lax.axis_index("c")
        s = lax.axis_index("s")
        wid = s * NC + c
        pltpu.sync_copy(idx_hbm.at[pl.ds(wid * TPW, TPW)], idx_v)

        def store_wait():
            pltpu.make_async_copy(
                rows_v.at[0], out_hbm.at[pl.ds(0, GROWS)], ssem).wait()

        def grp(g):
            for b in range(2):
                gg = 2 * g + b

                @pl.when(gg >= 2)
                def _():
                    store_wait()

                cps = []
                for j in range(GRP):
                    cps.append(pltpu.async_copy(
                        table_hbm.at[idx_v.at[gg * GRP + j]],
                        rows_v.at[b].at[pl.ds(j * CH, CH)], gsem))
                for cp in cps:
                    cp.wait()
                pltpu.async_copy(
                    rows_v.at[b],
                    out_hbm.at[pl.ds(wid * EPW + gg * GROWS, GROWS)], ssem)

        pl.loop(0, NGRP // 2)(grp)
        store_wait()
        store_wait()

    return pl.kernel(
        body,
        out_type=jax.ShapeDtypeStruct((EPAD, d), jnp.float32),
        mesh=mesh,
        scratch_types=[
            pltpu.VMEM((TPW, CH), jnp.int32),
            pltpu.VMEM((2, GROWS, d), jnp.float32),
            pltpu.SemaphoreType.DMA,
            pltpu.SemaphoreType.DMA,
        ],
        compiler_params=pltpu.CompilerParams(use_tc_tiling_on_sc=False),
        interpret=interpret,
    )


def _make_sc_scatter(interpret=False):
    mesh = plsc.VectorSubcoreMesh(core_axis_name="c", subcore_axis_name="s",
                                  num_cores=NC, num_subcores=NS)

    tpt = NROWS_IDX // NS

    def body(msg_hbm, idx_hbm, out_hbm, idx_v, msg_v, agg_sp,
             lsem, isem, ssem):
        c = lax.axis_index("c")
        s = lax.axis_index("s")

        z16 = jnp.zeros((16,), jnp.float32)

        def zrow(r):
            for q in range(FH // 16):
                msg_v[0, r, pl.ds(q * 16, 16)] = z16
                msg_v[1, r, pl.ds(q * 16, 16)] = z16

        pl.loop(0, CH)(zrow)

        def zcp(k):
            pltpu.sync_copy(msg_v.at[0],
                            agg_sp.at[pl.ds(s * STRIPE + k * CH, CH)])

        pl.loop(0, STRIPE // CH)(zcp)
        plsc.subcore_barrier()

        def scat_wait():
            pltpu.make_async_copy(
                msg_v.at[0], agg_sp.at[idx_v.at[0, 0]], ssem).wait()

        def grp(g):
            for b in range(2):
                gg = 2 * g + b

                @pl.when(gg >= 2)
                def _():
                    scat_wait()

                cpi = pltpu.async_copy(
                    idx_hbm.at[pl.ds(s * tpt + gg, 1)], idx_v.at[b], isem)
                cpm = pltpu.async_copy(
                    msg_hbm.at[c].at[pl.ds(s * tpt * CH + gg * CH, CH)],
                    msg_v.at[b], lsem)
                cpi.wait()
                cpm.wait()
                pltpu.async_copy(msg_v.at[b], agg_sp.at[idx_v.at[b, 0]],
                                 ssem, add=True)

        pl.loop(0, tpt // 2)(grp)
        scat_wait()
        scat_wait()
        plsc.subcore_barrier()

        pltpu.sync_copy(agg_sp.at[pl.ds(s * STRIPE, STRIPE)],
                        out_hbm.at[c, pl.ds(s * STRIPE, STRIPE)])

    return pl.kernel(
        body,
        out_type=jax.ShapeDtypeStruct((NC, NPAD, FH), jnp.float32),
        mesh=mesh,
        scratch_types=[
            pltpu.VMEM((2, 1, CH), jnp.int32),
            pltpu.VMEM((2, CH, FH), jnp.float32),
            pltpu.VMEM_SHARED((NPAD, FH), jnp.float32),
            pltpu.SemaphoreType.DMA,
            pltpu.SemaphoreType.DMA,
            pltpu.SemaphoreType.DMA,
        ],
        compiler_params=pltpu.CompilerParams(use_tc_tiling_on_sc=False),
        interpret=interpret,
    )


def _embed_body(z_ref, te_ref, o_ref):
    z = z_ref[0, 0, :]
    oh = (z[:, None] == lax.broadcasted_iota(jnp.int32, (BN, NTYPES), 1))
    o_ref[...] = jnp.dot(oh.astype(jnp.float32), te_ref[...],
                         preferred_element_type=jnp.float32)


def _tc_embed(z3, type_embed, interpret=False):
    return pl.pallas_call(
        _embed_body,
        grid=(NBLK_N,),
        in_specs=[
            pl.BlockSpec((1, 1, BN), lambda i: (i, 0, 0)),
            pl.BlockSpec((NTYPES, F), lambda i: (0, 0)),
        ],
        out_specs=pl.BlockSpec((BN, F), lambda i: (i, 0)),
        out_shape=jax.ShapeDtypeStruct((NPAD, F), jnp.float32),
        interpret=interpret,
    )(z3, type_embed)


def _edgefeat_body(ps_ref, pd_ref, o_ref):
    d = pd_ref[...] - ps_ref[...]
    colmask = (lax.broadcasted_iota(jnp.int32, (BEF, 16), 1) < 3)
    d2 = jnp.where(colmask, d * d, 0.0)
    r2 = jnp.sum(d2, axis=1, keepdims=True)
    r = jnp.sqrt(r2 + 1e-12)
    x = r / RMAX
    x2 = x * x
    x3 = x2 * x
    x6 = x3 * x3
    cut = 1.0 - x6 * (28.0 - 48.0 * x + 21.0 * x2)
    cut = jnp.where(x < 1.0, cut, 0.0)
    nf = (lax.broadcasted_iota(jnp.int32, (1, NBASIS), 1).astype(jnp.float32)
          + 1.0)
    rb = np.sqrt(2.0 / RMAX) * jnp.sin(nf * (np.pi / RMAX) * r) / (r + 1e-9)
    o_ref[...] = rb * cut


def _tc_edgefeat(ps, pd, interpret=False):
    return pl.pallas_call(
        _edgefeat_body,
        grid=(EPAD // BEF,),
        in_specs=[
            pl.BlockSpec((BEF, 16), lambda i: (i, 0)),
            pl.BlockSpec((BEF, 16), lambda i: (i, 0)),
        ],
        out_specs=pl.BlockSpec((BEF, NBASIS), lambda i: (i, 0)),
        out_shape=jax.ShapeDtypeStruct((EPAD, NBASIS), jnp.float32),
        interpret=interpret,
    )(ps, pd)


def _msg_body(ef_ref, hs_ref, w1_ref, b1_ref, w2_ref, b2_ref, o_ref):
    a = _silu(jnp.dot(ef_ref[...], w1_ref[...],
                      preferred_element_type=jnp.float32) + b1_ref[...])
    w = jnp.dot(a, w2_ref[...], preferred_element_type=jnp.float32) + b2_ref[...]
    m = w * hs_ref[...]
    o_ref[0, :, :] = m[:, :FH]
    o_ref[1, :, :] = m[:, FH:]


def _tc_msg(ef, hs, w1, b1, w2, b2, interpret=False):
    return pl.pallas_call(
        _msg_body,
        grid=(EPAD // BE,),
        in_specs=[
            pl.BlockSpec((BE, NBASIS), lambda i: (i, 0)),
            pl.BlockSpec((BE, F), lambda i: (i, 0)),
            pl.BlockSpec((NBASIS, HID), lambda i: (0, 0)),
            pl.BlockSpec((1, HID), lambda i: (0, 0)),
            pl.BlockSpec((HID, F), lambda i: (0, 0)),
            pl.BlockSpec((1, F), lambda i: (0, 0)),
        ],
        out_specs=pl.BlockSpec((NC, BE, FH), lambda i: (0, i, 0)),
        out_shape=jax.ShapeDtypeStruct((NC, EPAD, FH), jnp.float32),
        interpret=interpret,
    )(ef, hs, w1, b1, w2, b2)


def _hupd_body(h_ref, al_ref, ar_ref, ws_ref, wm_ref, o_ref):
    hp = jnp.dot(h_ref[...], ws_ref[...], preferred_element_type=jnp.float32)
    agg = jnp.concatenate([al_ref[0, :, :], ar_ref[0, :, :]], axis=1)
    ap = jnp.dot(agg * (1.0 / AVG_NEIGH), wm_ref[...],
                 preferred_element_type=jnp.float32)
    o_ref[...] = _silu(hp + ap)


def _tc_hupd(h, agg2, ws, wm, interpret=False):
    return pl.pallas_call(
        _hupd_body,
        grid=(NBLK_N,),
        in_specs=[
            pl.BlockSpec((BN, F), lambda i: (i, 0)),
            pl.BlockSpec((1, BN, FH), lambda i: (0, i, 0)),
            pl.BlockSpec((1, BN, FH), lambda i: (1, i, 0)),
            pl.BlockSpec((F, F), lambda i: (0, 0)),
            pl.BlockSpec((F, F), lambda i: (0, 0)),
        ],
        out_specs=pl.BlockSpec((BN, F), lambda i: (i, 0)),
        out_shape=jax.ShapeDtypeStruct((NPAD, F), jnp.float32),
        interpret=interpret,
    )(h, agg2, agg2, ws, wm)


def _read_body(h_ref, z_ref, w1_ref, b1_ref, w2_ref, b2_ref,
               sc_ref, sh_ref, o_ref):
    i = pl.program_id(0)
    s1 = _silu(jnp.dot(h_ref[...], w1_ref[...],
                       preferred_element_type=jnp.float32) + b1_ref[...])
    e = jnp.dot(s1, w2_ref[...], preferred_element_type=jnp.float32) + b2_ref[...]
    z = z_ref[0, 0, :]
    oh = (z[:, None] == lax.broadcasted_iota(jnp.int32, (BN, NTYPES), 1))
    ohf = oh.astype(jnp.float32)
    scv = jnp.sum(ohf * sc_ref[...], axis=1)
    shv = jnp.sum(ohf * sh_ref[...], axis=1)
    row = i * BN + lax.broadcasted_iota(jnp.int32, (BN,), 0)
    val = jnp.where(row < N, e[:, 0] * scv + shv, 0.0)

    @pl.when(i == 0)
    def _():
        o_ref[0, 0] = 0.0

    o_ref[0, 0] += jnp.sum(val)


def _tc_read(h, z3, w1, b1, w2, b2, sc, sh, interpret=False):
    return pl.pallas_call(
        _read_body,
        grid=(NBLK_N,),
        in_specs=[
            pl.BlockSpec((BN, F), lambda i: (i, 0)),
            pl.BlockSpec((1, 1, BN), lambda i: (i, 0, 0)),
            pl.BlockSpec((F, 32), lambda i: (0, 0)),
            pl.BlockSpec((1, 32), lambda i: (0, 0)),
            pl.BlockSpec((32, 1), lambda i: (0, 0)),
            pl.BlockSpec((1, 1), lambda i: (0, 0)),
            pl.BlockSpec((1, NTYPES), lambda i: (0, 0)),
            pl.BlockSpec((1, NTYPES), lambda i: (0, 0)),
        ],
        out_specs=pl.BlockSpec((1, 1), lambda i: (0, 0),
                               memory_space=pltpu.SMEM),
        out_shape=jax.ShapeDtypeStruct((1, 1), jnp.float32),
        interpret=interpret,
    )(h, z3, w1, b1, w2, b2, sc, sh)


def _run(pos, z, edge_index, type_embed, rW1, rb1, rW2, rb2, Wself, Wmsg,
         readW1, readb1, readW2, readb2, shifts, scales,
         interpret=False):
    src = edge_index[0].astype(jnp.int32)
    dst = edge_index[1].astype(jnp.int32)

    srcp = jnp.pad(src, (0, EPAD - E))
    dstp_g = jnp.pad(dst, (0, EPAD - E))
    dstp = jnp.pad(dst, (0, EPAD - E), constant_values=DUMP)
    src2d = srcp.reshape(NROWS_IDX, CH)
    dstg2d = dstp_g.reshape(NROWS_IDX, CH)
    dst2d = dstp.reshape(NROWS_IDX, CH)

    pos16 = jnp.pad(pos, ((0, 0), (0, 13)))
    zp = jnp.pad(z.astype(jnp.int32), (0, NPAD - N))
    z3 = zp.reshape(NBLK_N, 1, BN)

    gather16 = _make_sc_gather(16, interpret)
    gather64 = _make_sc_gather(F, interpret)
    scatter = _make_sc_scatter(interpret)

    ps = gather16(pos16, src2d)
    pd = gather16(pos16, dstg2d)
    ef = _tc_edgefeat(ps, pd, interpret)

    h = _tc_embed(z3, type_embed, interpret)
    for l in range(NLAYERS):
        hs = gather64(h, src2d)
        msg = _tc_msg(ef, hs, rW1[l], rb1[l].reshape(1, HID),
                      rW2[l], rb2[l].reshape(1, F), interpret)
        agg2 = scatter(msg, dst2d)
        h = _tc_hupd(h, agg2, Wself[l], Wmsg[l], interpret)

    tot = _tc_read(h, z3, readW1, readb1.reshape(1, 32),
                   readW2, readb2.reshape(1, 1),
                   scales.reshape(1, NTYPES), shifts.reshape(1, NTYPES),
                   interpret)
    return tot.reshape(1)


def kernel(pos, z, edge_index, type_embed, rW1, rb1, rW2, rb2, Wself, Wmsg,
           readW1, readb1, readW2, readb2, shifts, scales):
    return _run(pos, z, edge_index, type_embed, rW1, rb1, rW2, rb2,
                Wself, Wmsg, readW1, readb1, readW2, readb2, shifts, scales)


# speedup vs baseline: 1.0442x; 1.0442x over previous
import functools

import jax
import jax.numpy as jnp
import numpy as np
from jax import lax
from jax.experimental import pallas as pl
from jax.experimental.pallas import tpu as pltpu
from jax.experimental.pallas import tpu_sc as plsc

N = 50000
E = 800000
F = 64
NTYPES = 32
NBASIS = 8
NLAYERS = 4
RMAX = 5.0
AVG_NEIGH = 15.0
HID = 64

NC = 2
NS = 16
NW = NC * NS
CH = 256
GRP = 2
TPW = 100
NGRP = TPW // GRP
GROWS = GRP * CH
EPW = TPW * CH
EPAD = NW * EPW
NROWS_IDX = EPAD // CH

NPAD = 51200
FH = F // 2
STRIPE = NPAD // NS
DUMP = N + 1000
BN = 2048
NBLK_N = NPAD // BN
BEF = 4096
BE = 2048


def _silu(x):
    return x * jax.nn.sigmoid(x)


def _make_sc_gather(d, interpret=False):
    mesh = plsc.VectorSubcoreMesh(core_axis_name="c", subcore_axis_name="s",
                                  num_cores=NC, num_subcores=NS)

    def body(table_hbm, idx_hbm, out_hbm, idx_v, rows_v, gsem, ssem):
        c = lax.axis_index("c")
        s = lax.axis_index("s")
        wid = s * NC + c
        pltpu.sync_copy(idx_hbm.at[pl.ds(wid * TPW, TPW)], idx_v)

        def store_wait():
            pltpu.make_async_copy(
                rows_v.at[0], out_hbm.at[pl.ds(0, GROWS)], ssem).wait()

        def grp(g):
            for b in range(2):
                gg = 2 * g + b

                @pl.when(gg >= 2)
                def _():
                    store_wait()

                cps = []
                for j in range(GRP):
                    cps.append(pltpu.async_copy(
                        table_hbm.at[idx_v.at[gg * GRP + j]],
                        rows_v.at[b].at[pl.ds(j * CH, CH)], gsem))
                for cp in cps:
                    cp.wait()
                pltpu.async_copy(
                    rows_v.at[b],
                    out_hbm.at[pl.ds(wid * EPW + gg * GROWS, GROWS)], ssem)

        pl.loop(0, NGRP // 2)(grp)
        store_wait()
        store_wait()

    return pl.kernel(
        body,
        out_type=jax.ShapeDtypeStruct((EPAD, d), jnp.float32),
        mesh=mesh,
        scratch_types=[
            pltpu.VMEM((TPW, CH), jnp.int32),
            pltpu.VMEM((2, GROWS, d), jnp.float32),
            pltpu.SemaphoreType.DMA,
            pltpu.SemaphoreType.DMA,
        ],
        compiler_params=pltpu.CompilerParams(use_tc_tiling_on_sc=False),
        interpret=interpret,
    )


def _make_sc_scatter(interpret=False):
    mesh = plsc.VectorSubcoreMesh(core_axis_name="c", subcore_axis_name="s",
                                  num_cores=NC, num_subcores=NS)

    tpt = NROWS_IDX // NS

    def body(msg_hbm, idx_hbm, out_hbm, idx_v, msg_v, agg_sp,
             lsem, isem, ssem):
        c = lax.axis_index("c")
        s = lax.axis_index("s")

        z16 = jnp.zeros((16,), jnp.float32)

        def zrow(r):
            for q in range(FH // 16):
                msg_v[0, r, pl.ds(q * 16, 16)] = z16
                msg_v[1, r, pl.ds(q * 16, 16)] = z16

        pl.loop(0, CH)(zrow)

        def zcp(k):
            pltpu.sync_copy(msg_v.at[0],
                            agg_sp.at[pl.ds(s * STRIPE + k * CH, CH)])

        pl.loop(0, STRIPE // CH)(zcp)
        if STRIPE % CH:
            pltpu.sync_copy(
                msg_v.at[0].at[pl.ds(0, STRIPE % CH)],
                agg_sp.at[pl.ds(s * STRIPE + (STRIPE // CH) * CH,
                                STRIPE % CH)])
        plsc.subcore_barrier()

        def scat_wait():
            pltpu.make_async_copy(
                msg_v.at[0], agg_sp.at[idx_v.at[0, 0]], ssem).wait()

        def grp(g):
            for b in range(2):
                gg = 2 * g + b

                @pl.when(gg >= 2)
                def _():
                    scat_wait()

                cpi = pltpu.async_copy(
                    idx_hbm.at[pl.ds(s * tpt + gg, 1)], idx_v.at[b], isem)
                cpm = pltpu.async_copy(
                    msg_hbm.at[c].at[pl.ds(s * tpt * CH + gg * CH, CH)],
                    msg_v.at[b], lsem)
                cpi.wait()
                cpm.wait()
                pltpu.async_copy(msg_v.at[b], agg_sp.at[idx_v.at[b, 0]],
                                 ssem, add=True)

        pl.loop(0, tpt // 2)(grp)
        scat_wait()
        scat_wait()
        plsc.subcore_barrier()

        pltpu.sync_copy(agg_sp.at[pl.ds(s * STRIPE, STRIPE)],
                        out_hbm.at[c, pl.ds(s * STRIPE, STRIPE)])

    return pl.kernel(
        body,
        out_type=jax.ShapeDtypeStruct((NC, NPAD, FH), jnp.float32),
        mesh=mesh,
        scratch_types=[
            pltpu.VMEM((2, 1, CH), jnp.int32),
            pltpu.VMEM((2, CH, FH), jnp.float32),
            pltpu.VMEM_SHARED((NPAD, FH), jnp.float32),
            pltpu.SemaphoreType.DMA,
            pltpu.SemaphoreType.DMA,
            pltpu.SemaphoreType.DMA,
        ],
        compiler_params=pltpu.CompilerParams(use_tc_tiling_on_sc=False),
        interpret=interpret,
    )


def _embed_body(z_ref, te_ref, o_ref):
    z = z_ref[0, 0, :]
    oh = (z[:, None] == lax.broadcasted_iota(jnp.int32, (BN, NTYPES), 1))
    o_ref[...] = jnp.dot(oh.astype(jnp.float32), te_ref[...],
                         preferred_element_type=jnp.float32)


def _tc_embed(z3, type_embed, interpret=False):
    return pl.pallas_call(
        _embed_body,
        grid=(NBLK_N,),
        in_specs=[
            pl.BlockSpec((1, 1, BN), lambda i: (i, 0, 0)),
            pl.BlockSpec((NTYPES, F), lambda i: (0, 0)),
        ],
        out_specs=pl.BlockSpec((BN, F), lambda i: (i, 0)),
        out_shape=jax.ShapeDtypeStruct((NPAD, F), jnp.float32),
        interpret=interpret,
    )(z3, type_embed)


def _edgefeat_body(ps_ref, pd_ref, o_ref):
    d = pd_ref[...] - ps_ref[...]
    colmask = (lax.broadcasted_iota(jnp.int32, (BEF, 16), 1) < 3)
    d2 = jnp.where(colmask, d * d, 0.0)
    r2 = jnp.sum(d2, axis=1, keepdims=True)
    r = jnp.sqrt(r2 + 1e-12)
    x = r / RMAX
    x2 = x * x
    x3 = x2 * x
    x6 = x3 * x3
    cut = 1.0 - x6 * (28.0 - 48.0 * x + 21.0 * x2)
    cut = jnp.where(x < 1.0, cut, 0.0)
    nf = (lax.broadcasted_iota(jnp.int32, (1, NBASIS), 1).astype(jnp.float32)
          + 1.0)
    rb = np.sqrt(2.0 / RMAX) * jnp.sin(nf * (np.pi / RMAX) * r) / (r + 1e-9)
    o_ref[...] = rb * cut


def _tc_edgefeat(ps, pd, interpret=False):
    return pl.pallas_call(
        _edgefeat_body,
        grid=(EPAD // BEF,),
        in_specs=[
            pl.BlockSpec((BEF, 16), lambda i: (i, 0)),
            pl.BlockSpec((BEF, 16), lambda i: (i, 0)),
        ],
        out_specs=pl.BlockSpec((BEF, NBASIS), lambda i: (i, 0)),
        out_shape=jax.ShapeDtypeStruct((EPAD, NBASIS), jnp.float32),
        interpret=interpret,
    )(ps, pd)


def _msg_body(ef_ref, hs_ref, w1_ref, b1_ref, w2_ref, b2_ref, o_ref):
    a = _silu(jnp.dot(ef_ref[...], w1_ref[...],
                      preferred_element_type=jnp.float32) + b1_ref[...])
    w = jnp.dot(a, w2_ref[...], preferred_element_type=jnp.float32) + b2_ref[...]
    m = w * hs_ref[...]
    o_ref[0, :, :] = m[:, :FH]
    o_ref[1, :, :] = m[:, FH:]


def _tc_msg(ef, hs, w1, b1, w2, b2, interpret=False):
    return pl.pallas_call(
        _msg_body,
        grid=(EPAD // BE,),
        in_specs=[
            pl.BlockSpec((BE, NBASIS), lambda i: (i, 0)),
            pl.BlockSpec((BE, F), lambda i: (i, 0)),
            pl.BlockSpec((NBASIS, HID), lambda i: (0, 0)),
            pl.BlockSpec((1, HID), lambda i: (0, 0)),
            pl.BlockSpec((HID, F), lambda i: (0, 0)),
            pl.BlockSpec((1, F), lambda i: (0, 0)),
        ],
        out_specs=pl.BlockSpec((NC, BE, FH), lambda i: (0, i, 0)),
        out_shape=jax.ShapeDtypeStruct((NC, EPAD, FH), jnp.float32),
        interpret=interpret,
    )(ef, hs, w1, b1, w2, b2)


def _hupd_body(h_ref, al_ref, ar_ref, ws_ref, wm_ref, o_ref):
    hp = jnp.dot(h_ref[...], ws_ref[...], preferred_element_type=jnp.float32)
    agg = jnp.concatenate([al_ref[0, :, :], ar_ref[0, :, :]], axis=1)
    ap = jnp.dot(agg * (1.0 / AVG_NEIGH), wm_ref[...],
                 preferred_element_type=jnp.float32)
    o_ref[...] = _silu(hp + ap)


def _tc_hupd(h, agg2, ws, wm, interpret=False):
    return pl.pallas_call(
        _hupd_body,
        grid=(NBLK_N,),
        in_specs=[
            pl.BlockSpec((BN, F), lambda i: (i, 0)),
            pl.BlockSpec((1, BN, FH), lambda i: (0, i, 0)),
            pl.BlockSpec((1, BN, FH), lambda i: (1, i, 0)),
            pl.BlockSpec((F, F), lambda i: (0, 0)),
            pl.BlockSpec((F, F), lambda i: (0, 0)),
        ],
        out_specs=pl.BlockSpec((BN, F), lambda i: (i, 0)),
        out_shape=jax.ShapeDtypeStruct((NPAD, F), jnp.float32),
        interpret=interpret,
    )(h, agg2, agg2, ws, wm)


def _read_body(h_ref, z_ref, w1_ref, b1_ref, w2_ref, b2_ref,
               sc_ref, sh_ref, o_ref):
    i = pl.program_id(0)
    s1 = _silu(jnp.dot(h_ref[...], w1_ref[...],
                       preferred_element_type=jnp.float32) + b1_ref[...])
    e = jnp.dot(s1, w2_ref[...], preferred_element_type=jnp.float32) + b2_ref[...]
    z = z_ref[0, 0, :]
    oh = (z[:, None] == lax.broadcasted_iota(jnp.int32, (BN, NTYPES), 1))
    ohf = oh.astype(jnp.float32)
    scv = jnp.sum(ohf * sc_ref[...], axis=1)
    shv = jnp.sum(ohf * sh_ref[...], axis=1)
    row = i * BN + lax.broadcasted_iota(jnp.int32, (BN,), 0)
    val = jnp.where(row < N, e[:, 0] * scv + shv, 0.0)

    @pl.when(i == 0)
    def _():
        o_ref[0, 0] = 0.0

    o_ref[0, 0] += jnp.sum(val)


def _tc_read(h, z3, w1, b1, w2, b2, sc, sh, interpret=False):
    return pl.pallas_call(
        _read_body,
        grid=(NBLK_N,),
        in_specs=[
            pl.BlockSpec((BN, F), lambda i: (i, 0)),
            pl.BlockSpec((1, 1, BN), lambda i: (i, 0, 0)),
            pl.BlockSpec((F, 32), lambda i: (0, 0)),
            pl.BlockSpec((1, 32), lambda i: (0, 0)),
            pl.BlockSpec((32, 1), lambda i: (0, 0)),
            pl.BlockSpec((1, 1), lambda i: (0, 0)),
            pl.BlockSpec((1, NTYPES), lambda i: (0, 0)),
            pl.BlockSpec((1, NTYPES), lambda i: (0, 0)),
        ],
        out_specs=pl.BlockSpec((1, 1), lambda i: (0, 0),
                               memory_space=pltpu.SMEM),
        out_shape=jax.ShapeDtypeStruct((1, 1), jnp.float32),
        interpret=interpret,
    )(h, z3, w1, b1, w2, b2, sc, sh)


def _run(pos, z, edge_index, type_embed, rW1, rb1, rW2, rb2, Wself, Wmsg,
         readW1, readb1, readW2, readb2, shifts, scales,
         interpret=False):
    src = edge_index[0].astype(jnp.int32)
    dst = edge_index[1].astype(jnp.int32)

    srcp = jnp.pad(src, (0, EPAD - E))
    dstp_g = jnp.pad(dst, (0, EPAD - E))
    dstp = jnp.pad(dst, (0, EPAD - E), constant_values=DUMP)
    src2d = srcp.reshape(NROWS_IDX, CH)
    dstg2d = dstp_g.reshape(NROWS_IDX, CH)
    dst2d = dstp.reshape(NROWS_IDX, CH)

    pos16 = jnp.pad(pos, ((0, 0), (0, 13)))
    zp = jnp.pad(z.astype(jnp.int32), (0, NPAD - N))
    z3 = zp.reshape(NBLK_N, 1, BN)

    gather16 = _make_sc_gather(16, interpret)
    gather64 = _make_sc_gather(F, interpret)
    scatter = _make_sc_scatter(interpret)

    ps = gather16(pos16, src2d)
    pd = gather16(pos16, dstg2d)
    ef = _tc_edgefeat(ps, pd, interpret)

    h = _tc_embed(z3, type_embed, interpret)
    for l in range(NLAYERS):
        hs = gather64(h, src2d)
        msg = _tc_msg(ef, hs, rW1[l], rb1[l].reshape(1, HID),
                      rW2[l], rb2[l].reshape(1, F), interpret)
        agg2 = scatter(msg, dst2d)
        h = _tc_hupd(h, agg2, Wself[l], Wmsg[l], interpret)

    tot = _tc_read(h, z3, readW1, readb1.reshape(1, 32),
                   readW2, readb2.reshape(1, 1),
                   scales.reshape(1, NTYPES), shifts.reshape(1, NTYPES),
                   interpret)
    return tot.reshape(1)


def kernel(pos, z, edge_index, type_embed, rW1, rb1, rW2, rb2, Wself, Wmsg,
           readW1, readb1, readW2, readb2, shifts, scales):
    return _run(pos, z, edge_index, type_embed, rW1, rb1, rW2, rb2,
                Wself, Wmsg, readW1, readb1, readW2, readb2, shifts, scales)


# speedup vs baseline: 1.7275x; 1.6544x over previous
import functools

import jax
import jax.numpy as jnp
import numpy as np
from jax import lax
from jax.experimental import pallas as pl
from jax.experimental.pallas import tpu as pltpu
from jax.experimental.pallas import tpu_sc as plsc

N = 50000
E = 800000
F = 64
NTYPES = 32
NBASIS = 8
NLAYERS = 4
RMAX = 5.0
AVG_NEIGH = 15.0
HID = 64

NC = 2
NS = 16
NW = NC * NS
CH = 256
GRP = 2
TPW = 100
NGRP = TPW // GRP
GROWS = GRP * CH
EPW = TPW * CH
EPAD = NW * EPW
NROWS_IDX = EPAD // CH

NPAD = 51200
FH = F // 2
STRIPE = NPAD // NS
DUMP = N + 1000
BN = 2048
NBLK_N = NPAD // BN
BEF = 4096
BE = 2048


def _silu(x):
    return x * jax.nn.sigmoid(x)


def _make_sc_gather(d, interpret=False):
    mesh = plsc.VectorSubcoreMesh(core_axis_name="c", subcore_axis_name="s",
                                  num_cores=NC, num_subcores=NS)

    def body(table_hbm, idx_hbm, out_hbm, idx_v, rows_v, gsem, ssem):
        c = lax.axis_index("c")
        s = lax.axis_index("s")
        wid = s * NC + c
        pltpu.sync_copy(idx_hbm.at[pl.ds(wid * TPW, TPW)], idx_v)

        def store_wait():
            pltpu.make_async_copy(
                rows_v.at[0], out_hbm.at[pl.ds(0, GROWS)], ssem).wait()

        def grp(g):
            for b in range(2):
                gg = 2 * g + b

                @pl.when(gg >= 2)
                def _():
                    store_wait()

                cps = []
                for j in range(GRP):
                    cps.append(pltpu.async_copy(
                        table_hbm.at[idx_v.at[gg * GRP + j]],
                        rows_v.at[b].at[pl.ds(j * CH, CH)], gsem))
                for cp in cps:
                    cp.wait()
                pltpu.async_copy(
                    rows_v.at[b],
                    out_hbm.at[pl.ds(wid * EPW + gg * GROWS, GROWS)], ssem)

        pl.loop(0, NGRP // 2)(grp)
        store_wait()
        store_wait()

    return pl.kernel(
        body,
        out_type=jax.ShapeDtypeStruct((EPAD, d), jnp.float32),
        mesh=mesh,
        scratch_types=[
            pltpu.VMEM((TPW, CH), jnp.int32),
            pltpu.VMEM((2, GROWS, d), jnp.float32),
            pltpu.SemaphoreType.DMA,
            pltpu.SemaphoreType.DMA,
        ],
        compiler_params=pltpu.CompilerParams(use_tc_tiling_on_sc=False),
        interpret=interpret,
    )


def _make_sc_scatter(interpret=False):
    mesh = plsc.VectorSubcoreMesh(core_axis_name="c", subcore_axis_name="s",
                                  num_cores=NC, num_subcores=NS)

    tpt = NROWS_IDX // NS

    def body(msg_hbm, idx_hbm, out_hbm, idx_v, msg_v, agg_sp,
             lsem, isem, ssem):
        c = lax.axis_index("c")
        s = lax.axis_index("s")

        z16 = jnp.zeros((16,), jnp.float32)

        def zrow(r):
            for q in range(FH // 16):
                msg_v[0, r, pl.ds(q * 16, 16)] = z16
                msg_v[1, r, pl.ds(q * 16, 16)] = z16

        pl.loop(0, CH)(zrow)

        def zcp(k):
            pltpu.sync_copy(msg_v.at[0],
                            agg_sp.at[pl.ds(s * STRIPE + k * CH, CH)])

        pl.loop(0, STRIPE // CH)(zcp)
        if STRIPE % CH:
            pltpu.sync_copy(
                msg_v.at[0].at[pl.ds(0, STRIPE % CH)],
                agg_sp.at[pl.ds(s * STRIPE + (STRIPE // CH) * CH,
                                STRIPE % CH)])
        plsc.subcore_barrier()

        def scat_wait():
            pltpu.make_async_copy(
                msg_v.at[0], agg_sp.at[idx_v.at[0, 0]], ssem).wait()

        def grp(g):
            for b in range(2):
                gg = 2 * g + b

                @pl.when(gg >= 2)
                def _():
                    scat_wait()

                cpi = pltpu.async_copy(
                    idx_hbm.at[pl.ds(s * tpt + gg, 1)], idx_v.at[b], isem)
                cpm = pltpu.async_copy(
                    msg_hbm.at[c].at[pl.ds((s * tpt + gg) * CH, CH)],
                    msg_v.at[b], lsem)
                cpi.wait()
                cpm.wait()
                pltpu.async_copy(msg_v.at[b], agg_sp.at[idx_v.at[b, 0]],
                                 ssem, add=True)

        pl.loop(0, tpt // 2)(grp)
        scat_wait()
        scat_wait()
        plsc.subcore_barrier()

        pltpu.sync_copy(agg_sp.at[pl.ds(s * STRIPE, STRIPE)],
                        out_hbm.at[c, pl.ds(s * STRIPE, STRIPE)])

    return pl.kernel(
        body,
        out_type=jax.ShapeDtypeStruct((NC, NPAD, FH), jnp.float32),
        mesh=mesh,
        scratch_types=[
            pltpu.VMEM((2, 1, CH), jnp.int32),
            pltpu.VMEM((2, CH, FH), jnp.float32),
            pltpu.VMEM_SHARED((NPAD, FH), jnp.float32),
            pltpu.SemaphoreType.DMA,
            pltpu.SemaphoreType.DMA,
            pltpu.SemaphoreType.DMA,
        ],
        compiler_params=pltpu.CompilerParams(use_tc_tiling_on_sc=False),
        interpret=interpret,
    )


def _embed_body(z_ref, te_ref, o_ref):
    z = z_ref[0, 0, :]
    oh = (z[:, None] == lax.broadcasted_iota(jnp.int32, (BN, NTYPES), 1))
    o_ref[...] = jnp.dot(oh.astype(jnp.float32), te_ref[...],
                         preferred_element_type=jnp.float32)


def _tc_embed(z3, type_embed, interpret=False):
    return pl.pallas_call(
        _embed_body,
        grid=(NBLK_N,),
        in_specs=[
            pl.BlockSpec((1, 1, BN), lambda i: (i, 0, 0)),
            pl.BlockSpec((NTYPES, F), lambda i: (0, 0)),
        ],
        out_specs=pl.BlockSpec((BN, F), lambda i: (i, 0)),
        out_shape=jax.ShapeDtypeStruct((NPAD, F), jnp.float32),
        interpret=interpret,
    )(z3, type_embed)


_LANE = np.arange(128)
_SUMM = ((_LANE[:, None] // 16 == _LANE[None, :] // 16)
         & (_LANE[:, None] % 16 < 3)).astype(np.float32)
_J512 = np.arange(512)
_M1 = ((_LANE[:, None] // 16 == _J512[None, :] // 64)
       & (_LANE[:, None] % 16 < 8)).astype(np.float32)
_M2 = (_J512[:, None] // 64 == _J512[None, :] // 64).astype(np.float32)
_Q256 = np.arange(256)
_SELL = ((_J512[:, None]
          == (_Q256[None, :] // FH) * F + (_Q256[None, :] % FH))
         .astype(np.float32))
_SELR = ((_J512[:, None]
          == (_Q256[None, :] // FH) * F + FH + (_Q256[None, :] % FH))
         .astype(np.float32))


def _edgefeat_body(ps_ref, pd_ref, summ_ref, o_ref):
    d = pd_ref[...] - ps_ref[...]
    lane = lax.broadcasted_iota(jnp.int32, (BEF // 8, 128), 1)
    lm16 = lane % 16
    d2 = jnp.where(lm16 < 3, d * d, 0.0)
    r2 = jnp.dot(d2, summ_ref[...], preferred_element_type=jnp.float32)
    r = jnp.sqrt(r2 + 1e-12)
    x = r * (1.0 / RMAX)
    x2 = x * x
    x3 = x2 * x
    x6 = x3 * x3
    cut = 1.0 - x6 * (28.0 - 48.0 * x + 21.0 * x2)
    cut = jnp.where(x < 1.0, cut, 0.0)
    nf = (lm16 + 1).astype(jnp.float32)
    rb = np.sqrt(2.0 / RMAX) * jnp.sin(nf * (np.pi / RMAX) * r) / (r + 1e-9)
    o_ref[...] = jnp.where(lm16 < NBASIS, rb * cut, 0.0)


def _tc_edgefeat(ps, pd, interpret=False):
    return pl.pallas_call(
        _edgefeat_body,
        grid=(EPAD // BEF,),
        in_specs=[
            pl.BlockSpec((BEF // 8, 128), lambda i: (i, 0)),
            pl.BlockSpec((BEF // 8, 128), lambda i: (i, 0)),
            pl.BlockSpec((128, 128), lambda i: (0, 0)),
        ],
        out_specs=pl.BlockSpec((BEF // 8, 128), lambda i: (i, 0)),
        out_shape=jax.ShapeDtypeStruct((EPAD // 8, 128), jnp.float32),
        interpret=interpret,
    )(ps, pd, _SUMM)


def _msg_body(ef_ref, hs_ref, w1e_ref, b1e_ref, w2e_ref, b2e_ref,
              sell_ref, selr_ref, o_ref):
    a = _silu(jnp.dot(ef_ref[...], w1e_ref[...],
                      preferred_element_type=jnp.float32) + b1e_ref[...])
    w = jnp.dot(a, w2e_ref[...],
                preferred_element_type=jnp.float32) + b2e_ref[...]
    m = w * hs_ref[...]
    o_ref[0, :, :] = jnp.dot(m, sell_ref[...],
                             preferred_element_type=jnp.float32)
    o_ref[1, :, :] = jnp.dot(m, selr_ref[...],
                             preferred_element_type=jnp.float32)


def _tc_msg(ef, hs, w1e, b1e, w2e, b2e, interpret=False):
    return pl.pallas_call(
        _msg_body,
        grid=(EPAD // BE,),
        in_specs=[
            pl.BlockSpec((BE // 8, 128), lambda i: (i, 0)),
            pl.BlockSpec((BE // 8, 512), lambda i: (i, 0)),
            pl.BlockSpec((128, 512), lambda i: (0, 0)),
            pl.BlockSpec((1, 512), lambda i: (0, 0)),
            pl.BlockSpec((512, 512), lambda i: (0, 0)),
            pl.BlockSpec((1, 512), lambda i: (0, 0)),
            pl.BlockSpec((512, 256), lambda i: (0, 0)),
            pl.BlockSpec((512, 256), lambda i: (0, 0)),
        ],
        out_specs=pl.BlockSpec((NC, BE // 8, 256), lambda i: (0, i, 0)),
        out_shape=jax.ShapeDtypeStruct((NC, EPAD // 8, 256), jnp.float32),
        interpret=interpret,
    )(ef, hs, w1e, b1e, w2e, b2e, _SELL, _SELR)


def _hupd_body(h_ref, al_ref, ar_ref, ws_ref, wm_ref, o_ref):
    hp = jnp.dot(h_ref[...], ws_ref[...], preferred_element_type=jnp.float32)
    agg = jnp.concatenate([al_ref[0, :, :], ar_ref[0, :, :]], axis=1)
    ap = jnp.dot(agg * (1.0 / AVG_NEIGH), wm_ref[...],
                 preferred_element_type=jnp.float32)
    o_ref[...] = _silu(hp + ap)


def _tc_hupd(h, agg2, ws, wm, interpret=False):
    return pl.pallas_call(
        _hupd_body,
        grid=(NBLK_N,),
        in_specs=[
            pl.BlockSpec((BN, F), lambda i: (i, 0)),
            pl.BlockSpec((1, BN, FH), lambda i: (0, i, 0)),
            pl.BlockSpec((1, BN, FH), lambda i: (1, i, 0)),
            pl.BlockSpec((F, F), lambda i: (0, 0)),
            pl.BlockSpec((F, F), lambda i: (0, 0)),
        ],
        out_specs=pl.BlockSpec((BN, F), lambda i: (i, 0)),
        out_shape=jax.ShapeDtypeStruct((NPAD, F), jnp.float32),
        interpret=interpret,
    )(h, agg2, agg2, ws, wm)


def _read_body(h_ref, z_ref, w1_ref, b1_ref, w2_ref, b2_ref,
               sc_ref, sh_ref, o_ref):
    i = pl.program_id(0)
    s1 = _silu(jnp.dot(h_ref[...], w1_ref[...],
                       preferred_element_type=jnp.float32) + b1_ref[...])
    e = jnp.dot(s1, w2_ref[...], preferred_element_type=jnp.float32) + b2_ref[...]
    z = z_ref[0, 0, :]
    oh = (z[:, None] == lax.broadcasted_iota(jnp.int32, (BN, NTYPES), 1))
    ohf = oh.astype(jnp.float32)
    scv = jnp.sum(ohf * sc_ref[...], axis=1)
    shv = jnp.sum(ohf * sh_ref[...], axis=1)
    row = i * BN + lax.broadcasted_iota(jnp.int32, (BN,), 0)
    val = jnp.where(row < N, e[:, 0] * scv + shv, 0.0)

    @pl.when(i == 0)
    def _():
        o_ref[0, 0] = 0.0

    o_ref[0, 0] += jnp.sum(val)


def _tc_read(h, z3, w1, b1, w2, b2, sc, sh, interpret=False):
    return pl.pallas_call(
        _read_body,
        grid=(NBLK_N,),
        in_specs=[
            pl.BlockSpec((BN, F), lambda i: (i, 0)),
            pl.BlockSpec((1, 1, BN), lambda i: (i, 0, 0)),
            pl.BlockSpec((F, 32), lambda i: (0, 0)),
            pl.BlockSpec((1, 32), lambda i: (0, 0)),
            pl.BlockSpec((32, 1), lambda i: (0, 0)),
            pl.BlockSpec((1, 1), lambda i: (0, 0)),
            pl.BlockSpec((1, NTYPES), lambda i: (0, 0)),
            pl.BlockSpec((1, NTYPES), lambda i: (0, 0)),
        ],
        out_specs=pl.BlockSpec((1, 1), lambda i: (0, 0),
                               memory_space=pltpu.SMEM),
        out_shape=jax.ShapeDtypeStruct((1, 1), jnp.float32),
        interpret=interpret,
    )(h, z3, w1, b1, w2, b2, sc, sh)


def _run(pos, z, edge_index, type_embed, rW1, rb1, rW2, rb2, Wself, Wmsg,
         readW1, readb1, readW2, readb2, shifts, scales,
         interpret=False):
    src = edge_index[0].astype(jnp.int32)
    dst = edge_index[1].astype(jnp.int32)

    srcp = jnp.pad(src, (0, EPAD - E))
    dstp_g = jnp.pad(dst, (0, EPAD - E))
    dstp = jnp.pad(dst, (0, EPAD - E), constant_values=DUMP)
    src2d = srcp.reshape(NROWS_IDX, CH)
    dstg2d = dstp_g.reshape(NROWS_IDX, CH)
    dst2d = dstp.reshape(NROWS_IDX, CH)

    pos16 = jnp.pad(pos, ((0, 0), (0, 13)))
    zp = jnp.pad(z.astype(jnp.int32), (0, NPAD - N))
    z3 = zp.reshape(NBLK_N, 1, BN)

    gather16 = _make_sc_gather(16, interpret)
    gather64 = _make_sc_gather(F, interpret)
    scatter = _make_sc_scatter(interpret)

    ps = gather16(pos16, src2d).reshape(EPAD * 16 // 128, 128)
    pd = gather16(pos16, dstg2d).reshape(EPAD * 16 // 128, 128)
    ef = _tc_edgefeat(ps, pd, interpret)

    h = _tc_embed(z3, type_embed, interpret)
    for l in range(NLAYERS):
        hs = gather64(h, src2d).reshape(EPAD // 8, 512)
        w1e = jnp.tile(jnp.pad(rW1[l], ((0, 8), (0, 0))), (8, 8)) * _M1
        w2e = jnp.tile(rW2[l], (8, 8)) * _M2
        b1e = jnp.tile(rb1[l], 8).reshape(1, 512)
        b2e = jnp.tile(rb2[l], 8).reshape(1, 512)
        msg = _tc_msg(ef, hs, w1e, b1e, w2e, b2e, interpret)
        agg2 = scatter(msg.reshape(NC, EPAD, FH), dst2d)
        h = _tc_hupd(h, agg2, Wself[l], Wmsg[l], interpret)

    tot = _tc_read(h, z3, readW1, readb1.reshape(1, 32),
                   readW2, readb2.reshape(1, 1),
                   scales.reshape(1, NTYPES), shifts.reshape(1, NTYPES),
                   interpret)
    return tot.reshape(1)


def kernel(pos, z, edge_index, type_embed, rW1, rb1, rW2, rb2, Wself, Wmsg,
           readW1, readb1, readW2, readb2, shifts, scales):
    return _run(pos, z, edge_index, type_embed, rW1, rb1, rW2, rb2,
                Wself, Wmsg, readW1, readb1, readW2, readb2, shifts, scales)


# speedup vs baseline: 1.7307x; 1.0018x over previous
import functools

import jax
import jax.numpy as jnp
import numpy as np
from jax import lax
from jax.experimental import pallas as pl
from jax.experimental.pallas import tpu as pltpu
from jax.experimental.pallas import tpu_sc as plsc

N = 50000
E = 800000
F = 64
NTYPES = 32
NBASIS = 8
NLAYERS = 4
RMAX = 5.0
AVG_NEIGH = 15.0
HID = 64

NC = 2
NS = 16
NW = NC * NS
CH = 256
GRP = 2
TPW = 100
NGRP = TPW // GRP
GROWS = GRP * CH
EPW = TPW * CH
EPAD = NW * EPW
NROWS_IDX = EPAD // CH

NPAD = 51200
FH = F // 2
STRIPE = NPAD // NS
DUMP = N + 1000
BN = 2048
NBLK_N = NPAD // BN
BEF = 4096
BE = 2048


def _silu(x):
    return x * jax.nn.sigmoid(x)


def _make_sc_gather(d, interpret=False):
    mesh = plsc.VectorSubcoreMesh(core_axis_name="c", subcore_axis_name="s",
                                  num_cores=NC, num_subcores=NS)

    def body(table_hbm, idx_hbm, out_hbm, idx_v, rows_v, gsem, ssem):
        c = lax.axis_index("c")
        s = lax.axis_index("s")
        wid = s * NC + c
        pltpu.sync_copy(idx_hbm.at[pl.ds(wid * TPW, TPW)], idx_v)

        def store_wait():
            pltpu.make_async_copy(
                rows_v.at[0], out_hbm.at[pl.ds(0, GROWS)], ssem).wait()

        def grp(g):
            for b in range(2):
                gg = 2 * g + b

                @pl.when(gg >= 2)
                def _():
                    store_wait()

                cps = []
                for j in range(GRP):
                    cps.append(pltpu.async_copy(
                        table_hbm.at[idx_v.at[gg * GRP + j]],
                        rows_v.at[b].at[pl.ds(j * CH, CH)], gsem))
                for cp in cps:
                    cp.wait()
                pltpu.async_copy(
                    rows_v.at[b],
                    out_hbm.at[pl.ds(wid * EPW + gg * GROWS, GROWS)], ssem)

        pl.loop(0, NGRP // 2)(grp)
        store_wait()
        store_wait()

    return pl.kernel(
        body,
        out_type=jax.ShapeDtypeStruct((EPAD, d), jnp.float32),
        mesh=mesh,
        scratch_types=[
            pltpu.VMEM((TPW, CH), jnp.int32),
            pltpu.VMEM((2, GROWS, d), jnp.float32),
            pltpu.SemaphoreType.DMA,
            pltpu.SemaphoreType.DMA,
        ],
        compiler_params=pltpu.CompilerParams(use_tc_tiling_on_sc=False),
        interpret=interpret,
    )


def _make_sc_scatter(interpret=False):
    mesh = plsc.VectorSubcoreMesh(core_axis_name="c", subcore_axis_name="s",
                                  num_cores=NC, num_subcores=NS)

    tpt = NROWS_IDX // NS

    def body(msg_hbm, idx_hbm, out_hbm, idx_v, msg_v, agg_sp,
             lsem, isem, ssem):
        c = lax.axis_index("c")
        s = lax.axis_index("s")

        z16 = jnp.zeros((16,), jnp.float32)

        def zrow(r):
            for q in range(FH // 16):
                msg_v[0, r, pl.ds(q * 16, 16)] = z16
                msg_v[1, r, pl.ds(q * 16, 16)] = z16

        pl.loop(0, CH)(zrow)

        def zcp(k):
            pltpu.sync_copy(msg_v.at[0],
                            agg_sp.at[pl.ds(s * STRIPE + k * CH, CH)])

        pl.loop(0, STRIPE // CH)(zcp)
        if STRIPE % CH:
            pltpu.sync_copy(
                msg_v.at[0].at[pl.ds(0, STRIPE % CH)],
                agg_sp.at[pl.ds(s * STRIPE + (STRIPE // CH) * CH,
                                STRIPE % CH)])
        plsc.subcore_barrier()

        def scat_wait():
            pltpu.make_async_copy(
                msg_v.at[0], agg_sp.at[idx_v.at[0, 0]], ssem).wait()

        def grp(g):
            for b in range(2):
                gg = 2 * g + b

                @pl.when(gg >= 2)
                def _():
                    scat_wait()

                cpi = pltpu.async_copy(
                    idx_hbm.at[pl.ds(s * tpt + gg, 1)], idx_v.at[b], isem)
                cpm = pltpu.async_copy(
                    msg_hbm.at[c].at[pl.ds((s * tpt + gg) * CH, CH)],
                    msg_v.at[b], lsem)
                cpi.wait()
                cpm.wait()
                pltpu.async_copy(msg_v.at[b], agg_sp.at[idx_v.at[b, 0]],
                                 ssem, add=True)

        pl.loop(0, tpt // 2)(grp)
        scat_wait()
        scat_wait()
        plsc.subcore_barrier()

        pltpu.sync_copy(agg_sp.at[pl.ds(s * STRIPE, STRIPE)],
                        out_hbm.at[c, pl.ds(s * STRIPE, STRIPE)])

    return pl.kernel(
        body,
        out_type=jax.ShapeDtypeStruct((NC, NPAD, FH), jnp.float32),
        mesh=mesh,
        scratch_types=[
            pltpu.VMEM((2, 1, CH), jnp.int32),
            pltpu.VMEM((2, CH, FH), jnp.float32),
            pltpu.VMEM_SHARED((NPAD, FH), jnp.float32),
            pltpu.SemaphoreType.DMA,
            pltpu.SemaphoreType.DMA,
            pltpu.SemaphoreType.DMA,
        ],
        compiler_params=pltpu.CompilerParams(use_tc_tiling_on_sc=False),
        interpret=interpret,
    )


def _embed_body(z_ref, te_ref, o_ref):
    z = z_ref[0, 0, :]
    oh = (z[:, None] == lax.broadcasted_iota(jnp.int32, (BN, NTYPES), 1))
    o_ref[...] = jnp.dot(oh.astype(jnp.float32), te_ref[...],
                         preferred_element_type=jnp.float32)


def _tc_embed(z3, type_embed, interpret=False):
    return pl.pallas_call(
        _embed_body,
        grid=(NBLK_N,),
        in_specs=[
            pl.BlockSpec((1, 1, BN), lambda i: (i, 0, 0)),
            pl.BlockSpec((NTYPES, F), lambda i: (0, 0)),
        ],
        out_specs=pl.BlockSpec((BN, F), lambda i: (i, 0)),
        out_shape=jax.ShapeDtypeStruct((NPAD, F), jnp.float32),
        interpret=interpret,
    )(z3, type_embed)


_LANE = np.arange(128)
_SUMM = ((_LANE[:, None] // 16 == _LANE[None, :] // 16)
         & (_LANE[:, None] % 16 < 3)).astype(np.float32)
_J512 = np.arange(512)
_M1 = ((_LANE[:, None] // 16 == _J512[None, :] // 64)
       & (_LANE[:, None] % 16 < 8)).astype(np.float32)
_M2 = (_J512[:, None] // 64 == _J512[None, :] // 64).astype(np.float32)
_Q256 = np.arange(256)
_SELL = ((_J512[:, None]
          == (_Q256[None, :] // FH) * F + (_Q256[None, :] % FH))
         .astype(np.float32))
_SELR = ((_J512[:, None]
          == (_Q256[None, :] // FH) * F + FH + (_Q256[None, :] % FH))
         .astype(np.float32))
_M4 = (_Q256[:, None] // F == _Q256[None, :] // F).astype(np.float32)
_M4H = (_LANE[:, None] // FH == _Q256[None, :] // F).astype(np.float32)


def _edgefeat_body(ps_ref, pd_ref, summ_ref, o_ref):
    d = pd_ref[...] - ps_ref[...]
    lane = lax.broadcasted_iota(jnp.int32, (BEF // 8, 128), 1)
    lm16 = lane % 16
    d2 = jnp.where(lm16 < 3, d * d, 0.0)
    r2 = jnp.dot(d2, summ_ref[...], preferred_element_type=jnp.float32)
    r = jnp.sqrt(r2 + 1e-12)
    x = r * (1.0 / RMAX)
    x2 = x * x
    x3 = x2 * x
    x6 = x3 * x3
    cut = 1.0 - x6 * (28.0 - 48.0 * x + 21.0 * x2)
    cut = jnp.where(x < 1.0, cut, 0.0)
    nf = (lm16 + 1).astype(jnp.float32)
    rb = np.sqrt(2.0 / RMAX) * jnp.sin(nf * (np.pi / RMAX) * r) / (r + 1e-9)
    o_ref[...] = jnp.where(lm16 < NBASIS, rb * cut, 0.0)


def _tc_edgefeat(ps, pd, interpret=False):
    return pl.pallas_call(
        _edgefeat_body,
        grid=(EPAD // BEF,),
        in_specs=[
            pl.BlockSpec((BEF // 8, 128), lambda i: (i, 0)),
            pl.BlockSpec((BEF // 8, 128), lambda i: (i, 0)),
            pl.BlockSpec((128, 128), lambda i: (0, 0)),
        ],
        out_specs=pl.BlockSpec((BEF // 8, 128), lambda i: (i, 0)),
        out_shape=jax.ShapeDtypeStruct((EPAD // 8, 128), jnp.float32),
        interpret=interpret,
    )(ps, pd, _SUMM)


def _msg_body(ef_ref, hs_ref, w1e_ref, b1e_ref, w2e_ref, b2e_ref,
              sell_ref, selr_ref, o_ref):
    a = _silu(jnp.dot(ef_ref[...], w1e_ref[...],
                      preferred_element_type=jnp.float32) + b1e_ref[...])
    w = jnp.dot(a, w2e_ref[...],
                preferred_element_type=jnp.float32) + b2e_ref[...]
    m = w * hs_ref[...]
    o_ref[0, :, :] = jnp.dot(m, sell_ref[...],
                             preferred_element_type=jnp.float32)
    o_ref[1, :, :] = jnp.dot(m, selr_ref[...],
                             preferred_element_type=jnp.float32)


def _tc_msg(ef, hs, w1e, b1e, w2e, b2e, interpret=False):
    return pl.pallas_call(
        _msg_body,
        grid=(EPAD // BE,),
        in_specs=[
            pl.BlockSpec((BE // 8, 128), lambda i: (i, 0)),
            pl.BlockSpec((BE // 8, 512), lambda i: (i, 0)),
            pl.BlockSpec((128, 512), lambda i: (0, 0)),
            pl.BlockSpec((1, 512), lambda i: (0, 0)),
            pl.BlockSpec((512, 512), lambda i: (0, 0)),
            pl.BlockSpec((1, 512), lambda i: (0, 0)),
            pl.BlockSpec((512, 256), lambda i: (0, 0)),
            pl.BlockSpec((512, 256), lambda i: (0, 0)),
        ],
        out_specs=pl.BlockSpec((NC, BE // 8, 256), lambda i: (0, i, 0)),
        out_shape=jax.ShapeDtypeStruct((NC, EPAD // 8, 256), jnp.float32),
        interpret=interpret,
    )(ef, hs, w1e, b1e, w2e, b2e, _SELL, _SELR)


def _hupd_body(h4_ref, al_ref, ar_ref, wse_ref, wmle_ref, wmre_ref, o_ref):
    hp = jnp.dot(h4_ref[...], wse_ref[...],
                 preferred_element_type=jnp.float32)
    ap = (jnp.dot(al_ref[0, :, :], wmle_ref[...],
                  preferred_element_type=jnp.float32)
          + jnp.dot(ar_ref[0, :, :], wmre_ref[...],
                    preferred_element_type=jnp.float32))
    o_ref[...] = _silu(hp + ap * (1.0 / AVG_NEIGH))


def _tc_hupd(h4, agg2p, wse, wmle, wmre, interpret=False):
    return pl.pallas_call(
        _hupd_body,
        grid=(NBLK_N,),
        in_specs=[
            pl.BlockSpec((BN // 4, 256), lambda i: (i, 0)),
            pl.BlockSpec((1, BN // 4, 128), lambda i: (0, i, 0)),
            pl.BlockSpec((1, BN // 4, 128), lambda i: (1, i, 0)),
            pl.BlockSpec((256, 256), lambda i: (0, 0)),
            pl.BlockSpec((128, 256), lambda i: (0, 0)),
            pl.BlockSpec((128, 256), lambda i: (0, 0)),
        ],
        out_specs=pl.BlockSpec((BN // 4, 256), lambda i: (i, 0)),
        out_shape=jax.ShapeDtypeStruct((NPAD // 4, 256), jnp.float32),
        interpret=interpret,
    )(h4, agg2p, agg2p, wse, wmle, wmre)


def _read_body(h_ref, z_ref, w1_ref, b1_ref, w2_ref, b2_ref,
               sc_ref, sh_ref, o_ref):
    i = pl.program_id(0)
    s1 = _silu(jnp.dot(h_ref[...], w1_ref[...],
                       preferred_element_type=jnp.float32) + b1_ref[...])
    e = jnp.dot(s1, w2_ref[...], preferred_element_type=jnp.float32) + b2_ref[...]
    z = z_ref[0, 0, :]
    oh = (z[:, None] == lax.broadcasted_iota(jnp.int32, (BN, NTYPES), 1))
    ohf = oh.astype(jnp.float32)
    scv = jnp.sum(ohf * sc_ref[...], axis=1)
    shv = jnp.sum(ohf * sh_ref[...], axis=1)
    row = i * BN + lax.broadcasted_iota(jnp.int32, (BN,), 0)
    val = jnp.where(row < N, e[:, 0] * scv + shv, 0.0)

    @pl.when(i == 0)
    def _():
        o_ref[0, 0] = 0.0

    o_ref[0, 0] += jnp.sum(val)


def _tc_read(h, z3, w1, b1, w2, b2, sc, sh, interpret=False):
    return pl.pallas_call(
        _read_body,
        grid=(NBLK_N,),
        in_specs=[
            pl.BlockSpec((BN, F), lambda i: (i, 0)),
            pl.BlockSpec((1, 1, BN), lambda i: (i, 0, 0)),
            pl.BlockSpec((F, 32), lambda i: (0, 0)),
            pl.BlockSpec((1, 32), lambda i: (0, 0)),
            pl.BlockSpec((32, 1), lambda i: (0, 0)),
            pl.BlockSpec((1, 1), lambda i: (0, 0)),
            pl.BlockSpec((1, NTYPES), lambda i: (0, 0)),
            pl.BlockSpec((1, NTYPES), lambda i: (0, 0)),
        ],
        out_specs=pl.BlockSpec((1, 1), lambda i: (0, 0),
                               memory_space=pltpu.SMEM),
        out_shape=jax.ShapeDtypeStruct((1, 1), jnp.float32),
        interpret=interpret,
    )(h, z3, w1, b1, w2, b2, sc, sh)


def _run(pos, z, edge_index, type_embed, rW1, rb1, rW2, rb2, Wself, Wmsg,
         readW1, readb1, readW2, readb2, shifts, scales,
         interpret=False):
    src = edge_index[0].astype(jnp.int32)
    dst = edge_index[1].astype(jnp.int32)

    srcp = jnp.pad(src, (0, EPAD - E))
    dstp_g = jnp.pad(dst, (0, EPAD - E))
    dstp = jnp.pad(dst, (0, EPAD - E), constant_values=DUMP)
    src2d = srcp.reshape(NROWS_IDX, CH)
    dstg2d = dstp_g.reshape(NROWS_IDX, CH)
    dst2d = dstp.reshape(NROWS_IDX, CH)

    pos16 = jnp.pad(pos, ((0, 0), (0, 13)))
    zp = jnp.pad(z.astype(jnp.int32), (0, NPAD - N))
    z3 = zp.reshape(NBLK_N, 1, BN)

    gather16 = _make_sc_gather(16, interpret)
    gather64 = _make_sc_gather(F, interpret)
    scatter = _make_sc_scatter(interpret)

    ps = gather16(pos16, src2d).reshape(EPAD * 16 // 128, 128)
    pd = gather16(pos16, dstg2d).reshape(EPAD * 16 // 128, 128)
    ef = _tc_edgefeat(ps, pd, interpret)

    h = _tc_embed(z3, type_embed, interpret)
    for l in range(NLAYERS):
        hs = gather64(h, src2d).reshape(EPAD // 8, 512)
        w1e = jnp.tile(jnp.pad(rW1[l], ((0, 8), (0, 0))), (8, 8)) * _M1
        w2e = jnp.tile(rW2[l], (8, 8)) * _M2
        b1e = jnp.tile(rb1[l], 8).reshape(1, 512)
        b2e = jnp.tile(rb2[l], 8).reshape(1, 512)
        msg = _tc_msg(ef, hs, w1e, b1e, w2e, b2e, interpret)
        agg2 = scatter(msg.reshape(NC, EPAD, FH), dst2d)
        wse = jnp.tile(Wself[l], (4, 4)) * _M4
        wmle = jnp.tile(Wmsg[l][:FH], (4, 4)) * _M4H
        wmre = jnp.tile(Wmsg[l][FH:], (4, 4)) * _M4H
        h = _tc_hupd(h.reshape(NPAD // 4, 256),
                     agg2.reshape(NC, NPAD // 4, 128),
                     wse, wmle, wmre, interpret).reshape(NPAD, F)

    tot = _tc_read(h, z3, readW1, readb1.reshape(1, 32),
                   readW2, readb2.reshape(1, 1),
                   scales.reshape(1, NTYPES), shifts.reshape(1, NTYPES),
                   interpret)
    return tot.reshape(1)


def kernel(pos, z, edge_index, type_embed, rW1, rb1, rW2, rb2, Wself, Wmsg,
           readW1, readb1, readW2, readb2, shifts, scales):
    return _run(pos, z, edge_index, type_embed, rW1, rb1, rW2, rb2,
                Wself, Wmsg, readW1, readb1, readW2, readb2, shifts, scales)
